# baseline (speedup 1.0000x reference)
import jax
import jax.numpy as jnp
from jax import lax
from jax.experimental import pallas as pl
from jax.experimental.pallas import tpu as pltpu
from jax.experimental.pallas import tpu_sc as plsc

NC = 2
NS = 16
LANES = 16
NW = NC * NS

B = 4096
EMB = 128
LCODES = 50
MED = 1000

NBUF = 6
G = 2
GI = G * LCODES
NCH = EMB // LANES


def _make_emb(bh, off):
    bpw = bh // NW

    def body(dc_hbm, pc_hbm, dtab_hbm, ptab_hbm, out_hbm,
             idx_d, idx_p, rows, acc, sem):
        wid = lax.axis_index("s") * NC + lax.axis_index("c")
        base = off + wid * bpw
        pltpu.sync_copy(dc_hbm.at[pl.ds(base, bpw)], idx_d)
        pltpu.sync_copy(pc_hbm.at[pl.ds(base, bpw)], idx_p)

        def do_table(idx_v, tab_hbm, c0):
            for p in range(NBUF - 1):
                pltpu.async_copy(tab_hbm.at[idx_v.at[p]], rows.at[p], sem)

            def visit(v, carry):
                b = lax.rem(v, NBUF)
                pltpu.make_async_copy(
                    tab_hbm.at[idx_v.at[v]], rows.at[b], sem).wait()
                nxt = v + (NBUF - 1)

                @pl.when(nxt < bpw)
                def _():
                    pltpu.async_copy(
                        tab_hbm.at[idx_v.at[nxt]],
                        rows.at[lax.rem(nxt, NBUF)], sem)

                for h in range(4):
                    cs = [h * 2, h * 2 + 1]
                    accs = [rows[b, 0, pl.ds(c * LANES, LANES)]
                            for c in cs]
                    for i in range(1, LCODES):
                        for j, c in enumerate(cs):
                            accs[j] = accs[j] + rows[
                                b, i, pl.ds(c * LANES, LANES)]
                    for j, c in enumerate(cs):
                        acc[v, pl.ds(c0 + c * LANES, LANES)] = accs[j]
                return carry
            lax.fori_loop(0, bpw, visit, 0)

        do_table(idx_d, dtab_hbm, 0)
        do_table(idx_p, ptab_hbm, EMB)
        pltpu.sync_copy(acc, out_hbm.at[pl.ds(wid * bpw, bpw)])

    return pl.kernel(
        body,
        out_type=jax.ShapeDtypeStruct((bh, 2 * EMB), jnp.float32),
        mesh=plsc.VectorSubcoreMesh(
            core_axis_name="c", subcore_axis_name="s",
            num_cores=NC, num_subcores=NS),
        scratch_types=[
            pltpu.VMEM((bpw, LCODES), jnp.int32),
            pltpu.VMEM((bpw, LCODES), jnp.int32),
            pltpu.VMEM((NBUF, LCODES, EMB), jnp.float32),
            pltpu.VMEM((bpw, 2 * EMB), jnp.float32),
            pltpu.SemaphoreType.DMA,
        ],
    )


B1 = 3072
B2 = B - B1
_emb1 = _make_emb(B1, 0)
_emb2 = _make_emb(B2, B1)


def _mlp_body(comb_ref, w1_ref, b1_ref, w2_ref, b2_ref, out_ref):
    x = comb_ref[...]
    h = lax.dot_general(x, w1_ref[...], (((1,), (1,)), ((), ())),
                        preferred_element_type=jnp.float32)
    h = jnp.maximum(h + b1_ref[...], 0.0)
    z = lax.dot_general(h, w2_ref[...], (((1,), (1,)), ((), ())),
                        preferred_element_type=jnp.float32)
    z = z + b2_ref[...]
    out_ref[...] = 1.0 / (1.0 + jnp.exp(-z))


_ROWS_BLK = 1024


def _make_mlp(bh):
    return pl.pallas_call(
        _mlp_body,
        out_shape=jax.ShapeDtypeStruct((bh, MED), jnp.float32),
        grid=(bh // _ROWS_BLK,),
        in_specs=[
            pl.BlockSpec((_ROWS_BLK, 2 * EMB), lambda i: (i, 0)),
            pl.BlockSpec((64, 2 * EMB), lambda i: (0, 0)),
            pl.BlockSpec((1, 64), lambda i: (0, 0)),
            pl.BlockSpec((MED, 64), lambda i: (0, 0)),
            pl.BlockSpec((1, MED), lambda i: (0, 0)),
        ],
        out_specs=pl.BlockSpec((_ROWS_BLK, MED), lambda i: (i, 0)),
    )


_mlp1 = _make_mlp(B1)
_mlp2 = _make_mlp(B2)


@jax.jit
def kernel(diag_codes, proc_codes, diag_table, proc_table, W1, b1, W2, b2):
    b1r = b1.reshape(1, 64)
    b2r = b2.reshape(1, MED)
    comb1 = _emb1(diag_codes, proc_codes, diag_table, proc_table)
    comb2 = _emb2(diag_codes, proc_codes, diag_table, proc_table)
    out1 = _mlp1(comb1, W1, b1r, W2, b2r)
    out2 = _mlp2(comb2, W1, b1r, W2, b2r)
    return jnp.concatenate([out1, out2], axis=0)


# speedup vs baseline: 1.0279x; 1.0279x over previous
import jax
import jax.numpy as jnp
from jax import lax
from jax.experimental import pallas as pl
from jax.experimental.pallas import tpu as pltpu
from jax.experimental.pallas import tpu_sc as plsc

NC = 2
NS = 16
LANES = 16
NW = NC * NS

B = 4096
EMB = 128
LCODES = 50
MED = 1000

BPW = B // NW
G = 2
NG = BPW // G
GI = G * LCODES
NCH = EMB // LANES
NBUF = 3


def _emb_body(dc_hbm, pc_hbm, dtab_hbm, ptab_hbm, out_hbm,
              idx_d, idx_p, rows_d, rows_p, acc, sem_d, sem_p):
    wid = lax.axis_index("s") * NC + lax.axis_index("c")
    pltpu.sync_copy(dc_hbm.at[wid], idx_d)
    pltpu.sync_copy(pc_hbm.at[wid], idx_p)

    for p in range(NBUF - 1):
        pltpu.async_copy(dtab_hbm.at[idx_d.at[p]], rows_d.at[p], sem_d)
    for p in range(NBUF - 1):
        pltpu.async_copy(ptab_hbm.at[idx_p.at[p]], rows_p.at[p], sem_p)

    def do_table(idx_v, tab_hbm, rows, sem, c0):
        def group(g, carry):
            b = lax.rem(g, NBUF)
            pltpu.make_async_copy(
                tab_hbm.at[idx_v.at[g]], rows.at[b], sem).wait()
            nxt = g + (NBUF - 1)

            @pl.when(nxt < NG)
            def _():
                pltpu.async_copy(
                    tab_hbm.at[idx_v.at[nxt]],
                    rows.at[lax.rem(nxt, NBUF)], sem)

            for v in range(G):
                for h in range(4):
                    cs = [h * 2, h * 2 + 1]
                    accs = [rows[b, v * LCODES, pl.ds(c * LANES, LANES)]
                            for c in cs]
                    for i in range(1, LCODES):
                        for j, c in enumerate(cs):
                            accs[j] = accs[j] + rows[b, v * LCODES + i,
                                                     pl.ds(c * LANES, LANES)]
                    for j, c in enumerate(cs):
                        acc[g * G + v, pl.ds(c0 + c * LANES, LANES)] = accs[j]
            return carry
        lax.fori_loop(0, NG, group, 0)

    do_table(idx_d, dtab_hbm, rows_d, sem_d, 0)
    do_table(idx_p, ptab_hbm, rows_p, sem_p, EMB)
    pltpu.sync_copy(acc, out_hbm.at[wid])


_emb = pl.kernel(
    _emb_body,
    out_type=jax.ShapeDtypeStruct((NW, BPW, 2 * EMB), jnp.float32),
    mesh=plsc.VectorSubcoreMesh(
        core_axis_name="c", subcore_axis_name="s",
        num_cores=NC, num_subcores=NS),
    scratch_types=[
        pltpu.VMEM((NG, GI), jnp.int32),
        pltpu.VMEM((NG, GI), jnp.int32),
        pltpu.VMEM((NBUF, GI, EMB), jnp.float32),
        pltpu.VMEM((NBUF, GI, EMB), jnp.float32),
        pltpu.VMEM((BPW, 2 * EMB), jnp.float32),
        pltpu.SemaphoreType.DMA,
        pltpu.SemaphoreType.DMA,
    ],
)


def _mlp_body(comb_ref, w1_ref, b1_ref, w2_ref, b2_ref, out_ref):
    x = comb_ref[...].astype(jnp.bfloat16)
    w1 = w1_ref[...].astype(jnp.bfloat16)
    h = lax.dot_general(x, w1, (((1,), (1,)), ((), ())),
                        preferred_element_type=jnp.float32)
    h = jnp.maximum(h + b1_ref[...], 0.0)
    w2 = w2_ref[...].astype(jnp.bfloat16)
    z = lax.dot_general(h.astype(jnp.bfloat16), w2,
                        (((1,), (1,)), ((), ())),
                        preferred_element_type=jnp.float32)
    z = z + b2_ref[...]
    out_ref[...] = 1.0 / (1.0 + jnp.exp(-z))


_ROWS_BLK = 1024

_mlp = pl.pallas_call(
    _mlp_body,
    out_shape=jax.ShapeDtypeStruct((B, MED), jnp.float32),
    grid=(B // _ROWS_BLK,),
    in_specs=[
        pl.BlockSpec((_ROWS_BLK, 2 * EMB), lambda i: (i, 0)),
        pl.BlockSpec((64, 2 * EMB), lambda i: (0, 0)),
        pl.BlockSpec((1, 64), lambda i: (0, 0)),
        pl.BlockSpec((MED, 64), lambda i: (0, 0)),
        pl.BlockSpec((1, MED), lambda i: (0, 0)),
    ],
    out_specs=pl.BlockSpec((_ROWS_BLK, MED), lambda i: (i, 0)),
)


@jax.jit
def kernel(diag_codes, proc_codes, diag_table, proc_table, W1, b1, W2, b2):
    dc = diag_codes.reshape(NW, NG, GI)
    pc = proc_codes.reshape(NW, NG, GI)
    comb = _emb(dc, pc, diag_table, proc_table).reshape(B, 2 * EMB)
    return _mlp(comb, W1, b1.reshape(1, 64), W2, b2.reshape(1, MED))


# speedup vs baseline: 1.1603x; 1.1288x over previous
import functools

import jax
import jax.numpy as jnp
from jax import lax
from jax.experimental import pallas as pl
from jax.experimental.pallas import tpu as pltpu
from jax.experimental.pallas import tpu_sc as plsc

NC = 2
NS = 16
LANES = 16
NW = NC * NS

B = 4096
EMB = 128
LCODES = 50
MED = 1000

BPW = B // NW
G = 2
NG = BPW // G
GI = G * LCODES


NBUF = 6


def _emb_body(dc_hbm, pc_hbm, dtab_hbm, ptab_hbm, out_hbm,
              idx_d, idx_p, rows, acc, sem):
    wid = lax.axis_index("s") * NC + lax.axis_index("c")
    pltpu.sync_copy(dc_hbm.at[wid], idx_d)
    pltpu.sync_copy(pc_hbm.at[wid], idx_p)

    def do_table(idx_v, tab_hbm, c0):
        for p in range(NBUF - 1):
            pltpu.async_copy(tab_hbm.at[idx_v.at[p]], rows.at[p], sem)

        def group(g, carry):
            b = lax.rem(g, NBUF)
            pltpu.make_async_copy(
                tab_hbm.at[idx_v.at[g]], rows.at[b], sem).wait()
            nxt = g + (NBUF - 1)

            @pl.when(nxt < NG)
            def _():
                pltpu.async_copy(
                    tab_hbm.at[idx_v.at[nxt]],
                    rows.at[lax.rem(nxt, NBUF)], sem)

            NCH = EMB // LANES
            HALF = NCH // 4
            for v in range(G):
                for h in range(4):
                    cs = [h * HALF + c for c in range(HALF)]
                    accs = [rows[b, v * LCODES, pl.ds(c * LANES, LANES)]
                            for c in cs]
                    for i in range(1, LCODES):
                        for j, c in enumerate(cs):
                            accs[j] = accs[j] + rows[b, v * LCODES + i,
                                                     pl.ds(c * LANES, LANES)]
                    for j, c in enumerate(cs):
                        acc[g * G + v, pl.ds(c0 + c * LANES, LANES)] = accs[j]
            return carry
        lax.fori_loop(0, NG, group, 0)

    do_table(idx_d, dtab_hbm, 0)
    do_table(idx_p, ptab_hbm, EMB)
    pltpu.sync_copy(acc, out_hbm.at[wid])


_emb = pl.kernel(
    _emb_body,
    out_type=jax.ShapeDtypeStruct((NW, BPW, 2 * EMB), jnp.float32),
    mesh=plsc.VectorSubcoreMesh(
        core_axis_name="c", subcore_axis_name="s",
        num_cores=NC, num_subcores=NS),
    scratch_types=[
        pltpu.VMEM((NG, GI), jnp.int32),
        pltpu.VMEM((NG, GI), jnp.int32),
        pltpu.VMEM((NBUF, GI, EMB), jnp.float32),
        pltpu.VMEM((BPW, 2 * EMB), jnp.float32),
        pltpu.SemaphoreType.DMA,
    ],
)


def _mlp_body(comb_ref, w1t_ref, b1_ref, w2t_ref, b2_ref, out_ref):
    x = comb_ref[...]
    h = jnp.dot(x, w1t_ref[...], preferred_element_type=jnp.float32)
    h = jnp.maximum(h + b1_ref[...], 0.0)
    z = jnp.dot(h, w2t_ref[...], preferred_element_type=jnp.float32)
    z = z + b2_ref[...]
    out_ref[...] = 1.0 / (1.0 + jnp.exp(-z))


_ROWS_BLK = 1024

_mlp = pl.pallas_call(
    _mlp_body,
    out_shape=jax.ShapeDtypeStruct((B, MED), jnp.float32),
    grid=(B // _ROWS_BLK,),
    in_specs=[
        pl.BlockSpec((_ROWS_BLK, 2 * EMB), lambda i: (i, 0)),
        pl.BlockSpec((2 * EMB, 64), lambda i: (0, 0)),
        pl.BlockSpec((1, 64), lambda i: (0, 0)),
        pl.BlockSpec((64, MED), lambda i: (0, 0)),
        pl.BlockSpec((1, MED), lambda i: (0, 0)),
    ],
    out_specs=pl.BlockSpec((_ROWS_BLK, MED), lambda i: (i, 0)),
)


@jax.jit
def kernel(diag_codes, proc_codes, diag_table, proc_table, W1, b1, W2, b2):
    dc = diag_codes.reshape(NW, NG, GI)
    pc = proc_codes.reshape(NW, NG, GI)
    comb = _emb(dc, pc, diag_table, proc_table).reshape(B, 2 * EMB)
    return _mlp(comb, W1.T, b1.reshape(1, 64), W2.T, b2.reshape(1, MED))
